# baseline (speedup 1.0000x reference)
import functools

import jax
import jax.numpy as jnp
from jax import lax
from jax.experimental import pallas as pl
from jax.experimental.pallas import tpu as pltpu
from jax.experimental.pallas import tpu_sc as plsc

_B = 64
_C = 384
_ROWS_PER_WORKER = 8
_NUM_ACTIVE = 16


def _sc_gather(y, gammas_table, betas_table):
    mesh = plsc.VectorSubcoreMesh(core_axis_name="c", subcore_axis_name="s")

    @functools.partial(
        pl.kernel,
        out_type=[
            jax.ShapeDtypeStruct((_B, _C), jnp.float32),
            jax.ShapeDtypeStruct((_B, _C), jnp.float32),
        ],
        mesh=mesh,
        scratch_types=[
            pltpu.VMEM((_ROWS_PER_WORKER,), jnp.int32),
            pltpu.VMEM((_ROWS_PER_WORKER, _C), jnp.float32),
            pltpu.SemaphoreType.DMA,
        ],
    )
    def gather_kernel(y_hbm, gt_hbm, bt_hbm, g_out, b_out, idx_v, rows_v, sem):
        wid = lax.axis_index("s") * 2 + lax.axis_index("c")
        base = (wid % 8) * _ROWS_PER_WORKER

        @pl.when(wid < 8)
        def _():
            pltpu.sync_copy(y_hbm.at[pl.ds(base, _ROWS_PER_WORKER)], idx_v)
            pltpu.async_copy(gt_hbm.at[idx_v], rows_v, sem).wait()
            pltpu.sync_copy(rows_v, g_out.at[pl.ds(base, _ROWS_PER_WORKER)])

        @pl.when((wid >= 8) & (wid < _NUM_ACTIVE))
        def _():
            pltpu.sync_copy(y_hbm.at[pl.ds(base, _ROWS_PER_WORKER)], idx_v)
            pltpu.async_copy(bt_hbm.at[idx_v], rows_v, sem).wait()
            pltpu.sync_copy(rows_v, b_out.at[pl.ds(base, _ROWS_PER_WORKER)])

    return gather_kernel(y, gammas_table, betas_table)


def _film_body(x_ref, g_ref, b_ref, o_ref):
    g = g_ref[...][:, 0, :, None, None]
    b = b_ref[...][:, 0, :, None, None]
    o_ref[...] = x_ref[...] * g + b


def _film(x, g, b, cb=128):
    bsz, c, h, w = x.shape
    grid = (bsz, c // cb)
    g3 = g.reshape(bsz, 1, c)
    b3 = b.reshape(bsz, 1, c)
    return pl.pallas_call(
        _film_body,
        grid=grid,
        in_specs=[
            pl.BlockSpec((1, cb, h, w), lambda i, j: (i, j, 0, 0)),
            pl.BlockSpec((1, 1, cb), lambda i, j: (i, 0, j)),
            pl.BlockSpec((1, 1, cb), lambda i, j: (i, 0, j)),
        ],
        out_specs=pl.BlockSpec((1, cb, h, w), lambda i, j: (i, j, 0, 0)),
        out_shape=jax.ShapeDtypeStruct((bsz, c, h, w), x.dtype),
        compiler_params=pltpu.CompilerParams(
            dimension_semantics=("parallel", "parallel"),
        ),
    )(x, g3, b3)


def kernel(x, y, gammas_table, betas_table):
    g, b = _sc_gather(y.astype(jnp.int32), gammas_table, betas_table)
    out = _film(x, g, b)
    return (out, y)


# speedup vs baseline: 8.2717x; 8.2717x over previous
import functools

import jax
import jax.numpy as jnp
from jax import lax
from jax.experimental import pallas as pl
from jax.experimental.pallas import tpu as pltpu
from jax.experimental.pallas import tpu_sc as plsc

_B = 64
_C = 384
_ROWS_PER_WORKER = 8
_NUM_ACTIVE = 16


def _sc_gather(y, gammas_table, betas_table):
    mesh = plsc.VectorSubcoreMesh(core_axis_name="c", subcore_axis_name="s")

    @functools.partial(
        pl.kernel,
        out_type=[
            jax.ShapeDtypeStruct((_B, _C), jnp.float32),
            jax.ShapeDtypeStruct((_B, _C), jnp.float32),
        ],
        mesh=mesh,
        scratch_types=[
            pltpu.VMEM((_ROWS_PER_WORKER,), jnp.int32),
            pltpu.VMEM((_ROWS_PER_WORKER, _C), jnp.float32),
            pltpu.SemaphoreType.DMA,
        ],
    )
    def gather_kernel(y_hbm, gt_hbm, bt_hbm, g_out, b_out, idx_v, rows_v, sem):
        wid = lax.axis_index("s") * 2 + lax.axis_index("c")
        base = (wid % 8) * _ROWS_PER_WORKER

        @pl.when(wid < 8)
        def _():
            pltpu.sync_copy(y_hbm.at[pl.ds(base, _ROWS_PER_WORKER)], idx_v)
            pltpu.async_copy(gt_hbm.at[idx_v], rows_v, sem).wait()
            pltpu.sync_copy(rows_v, g_out.at[pl.ds(base, _ROWS_PER_WORKER)])

        @pl.when((wid >= 8) & (wid < _NUM_ACTIVE))
        def _():
            pltpu.sync_copy(y_hbm.at[pl.ds(base, _ROWS_PER_WORKER)], idx_v)
            pltpu.async_copy(bt_hbm.at[idx_v], rows_v, sem).wait()
            pltpu.sync_copy(rows_v, b_out.at[pl.ds(base, _ROWS_PER_WORKER)])

    return gather_kernel(y, gammas_table, betas_table)


def _film_body(x_ref, g_ref, b_ref, o_ref):
    o_ref[...] = x_ref[...] * g_ref[...] + b_ref[...]


def _film_planes(xt, g, b, pb=16):
    p, bsz, c = xt.shape
    grid = (p // pb,)
    return pl.pallas_call(
        _film_body,
        grid=grid,
        in_specs=[
            pl.BlockSpec((pb, bsz, c), lambda i: (i, 0, 0)),
            pl.BlockSpec((bsz, c), lambda i: (0, 0)),
            pl.BlockSpec((bsz, c), lambda i: (0, 0)),
        ],
        out_specs=pl.BlockSpec((pb, bsz, c), lambda i: (i, 0, 0)),
        out_shape=jax.ShapeDtypeStruct((p, bsz, c), xt.dtype),
        compiler_params=pltpu.CompilerParams(
            dimension_semantics=("arbitrary",),
        ),
    )(xt, g, b)


def kernel(x, y, gammas_table, betas_table):
    g, b = _sc_gather(y.astype(jnp.int32), gammas_table, betas_table)
    bsz, c, h, w = x.shape
    xt = jnp.transpose(x, (2, 3, 0, 1)).reshape(h * w, bsz, c)
    ot = _film_planes(xt, g, b)
    out = jnp.transpose(ot.reshape(h, w, bsz, c), (2, 3, 0, 1))
    return (out, y)


# speedup vs baseline: 8.2906x; 1.0023x over previous
import functools

import jax
import jax.numpy as jnp
from jax import lax
from jax.experimental import pallas as pl
from jax.experimental.pallas import tpu as pltpu
from jax.experimental.pallas import tpu_sc as plsc

_B = 64
_C = 384
_ROWS_PER_WORKER = 8
_NUM_ACTIVE = 16


def _sc_gather(y, gammas_table, betas_table):
    mesh = plsc.VectorSubcoreMesh(core_axis_name="c", subcore_axis_name="s")

    @functools.partial(
        pl.kernel,
        out_type=[
            jax.ShapeDtypeStruct((_B, _C), jnp.float32),
            jax.ShapeDtypeStruct((_B, _C), jnp.float32),
        ],
        mesh=mesh,
        scratch_types=[
            pltpu.VMEM((_ROWS_PER_WORKER,), jnp.int32),
            pltpu.VMEM((_ROWS_PER_WORKER, _C), jnp.float32),
            pltpu.SemaphoreType.DMA,
        ],
    )
    def gather_kernel(y_hbm, gt_hbm, bt_hbm, g_out, b_out, idx_v, rows_v, sem):
        wid = lax.axis_index("s") * 2 + lax.axis_index("c")
        base = (wid % 8) * _ROWS_PER_WORKER

        @pl.when(wid < 8)
        def _():
            pltpu.sync_copy(y_hbm.at[pl.ds(base, _ROWS_PER_WORKER)], idx_v)
            pltpu.async_copy(gt_hbm.at[idx_v], rows_v, sem).wait()
            pltpu.sync_copy(rows_v, g_out.at[pl.ds(base, _ROWS_PER_WORKER)])

        @pl.when((wid >= 8) & (wid < _NUM_ACTIVE))
        def _():
            pltpu.sync_copy(y_hbm.at[pl.ds(base, _ROWS_PER_WORKER)], idx_v)
            pltpu.async_copy(bt_hbm.at[idx_v], rows_v, sem).wait()
            pltpu.sync_copy(rows_v, b_out.at[pl.ds(base, _ROWS_PER_WORKER)])

    return gather_kernel(y, gammas_table, betas_table)


def _film_body(x_ref, g_ref, b_ref, o_ref):
    o_ref[...] = x_ref[...] * g_ref[...] + b_ref[...]


def _film_planes(xt, g, b, pb=16):
    p, bsz, c = xt.shape
    grid = (p // pb,)
    return pl.pallas_call(
        _film_body,
        grid=grid,
        in_specs=[
            pl.BlockSpec((pb, bsz, c), lambda i: (i, 0, 0)),
            pl.BlockSpec((bsz, c), lambda i: (0, 0)),
            pl.BlockSpec((bsz, c), lambda i: (0, 0)),
        ],
        out_specs=pl.BlockSpec((pb, bsz, c), lambda i: (i, 0, 0)),
        out_shape=jax.ShapeDtypeStruct((p, bsz, c), xt.dtype),
        compiler_params=pltpu.CompilerParams(
            dimension_semantics=("parallel",),
        ),
    )(xt, g, b)


def kernel(x, y, gammas_table, betas_table):
    g, b = _sc_gather(y.astype(jnp.int32), gammas_table, betas_table)
    bsz, c, h, w = x.shape
    xt = jnp.transpose(x, (2, 3, 0, 1)).reshape(h * w, bsz, c)
    ot = _film_planes(xt, g, b)
    out = jnp.transpose(ot.reshape(h, w, bsz, c), (2, 3, 0, 1))
    return (out, y)


# speedup vs baseline: 9.6363x; 1.1623x over previous
import functools

import jax
import jax.numpy as jnp
from jax import lax
from jax.experimental import pallas as pl
from jax.experimental.pallas import tpu as pltpu
from jax.experimental.pallas import tpu_sc as plsc

_B = 64
_C = 384
_ROWS_PER_WORKER = 8
_NUM_ACTIVE = 16


def _sc_gather(y, gammas_table, betas_table):
    mesh = plsc.VectorSubcoreMesh(core_axis_name="c", subcore_axis_name="s")

    @functools.partial(
        pl.kernel,
        out_type=[
            jax.ShapeDtypeStruct((_B, _C), jnp.float32),
            jax.ShapeDtypeStruct((_B, _C), jnp.float32),
        ],
        mesh=mesh,
        scratch_types=[
            pltpu.VMEM((_ROWS_PER_WORKER,), jnp.int32),
            pltpu.VMEM((_ROWS_PER_WORKER, _C), jnp.float32),
            pltpu.SemaphoreType.DMA,
        ],
    )
    def gather_kernel(y_hbm, gt_hbm, bt_hbm, g_out, b_out, idx_v, rows_v, sem):
        wid = lax.axis_index("s") * 2 + lax.axis_index("c")
        base = (wid % 8) * _ROWS_PER_WORKER

        @pl.when(wid < 8)
        def _():
            pltpu.sync_copy(y_hbm.at[pl.ds(base, _ROWS_PER_WORKER)], idx_v)
            pltpu.async_copy(gt_hbm.at[idx_v], rows_v, sem).wait()
            pltpu.sync_copy(rows_v, g_out.at[pl.ds(base, _ROWS_PER_WORKER)])

        @pl.when((wid >= 8) & (wid < _NUM_ACTIVE))
        def _():
            pltpu.sync_copy(y_hbm.at[pl.ds(base, _ROWS_PER_WORKER)], idx_v)
            pltpu.async_copy(bt_hbm.at[idx_v], rows_v, sem).wait()
            pltpu.sync_copy(rows_v, b_out.at[pl.ds(base, _ROWS_PER_WORKER)])

    return gather_kernel(y, gammas_table, betas_table)


def _film_body(x_ref, g_ref, b_ref, o_ref):
    o_ref[...] = x_ref[...] * g_ref[...] + b_ref[...]


def _film_planes(xt, g, b, pb=49):
    p, bsz, c = xt.shape
    grid = (p // pb,)
    return pl.pallas_call(
        _film_body,
        grid=grid,
        in_specs=[
            pl.BlockSpec((pb, bsz, c), lambda i: (i, 0, 0)),
            pl.BlockSpec((bsz, c), lambda i: (0, 0)),
            pl.BlockSpec((bsz, c), lambda i: (0, 0)),
        ],
        out_specs=pl.BlockSpec((pb, bsz, c), lambda i: (i, 0, 0)),
        out_shape=jax.ShapeDtypeStruct((p, bsz, c), xt.dtype),
        compiler_params=pltpu.CompilerParams(
            dimension_semantics=("parallel",),
        ),
    )(xt, g, b)


def kernel(x, y, gammas_table, betas_table):
    g, b = _sc_gather(y.astype(jnp.int32), gammas_table, betas_table)
    bsz, c, h, w = x.shape
    xt = jnp.transpose(x, (2, 3, 0, 1)).reshape(h * w, bsz, c)
    ot = _film_planes(xt, g, b)
    out = jnp.transpose(ot.reshape(h, w, bsz, c), (2, 3, 0, 1))
    return (out, y)


# speedup vs baseline: 9.8665x; 1.0239x over previous
import functools

import jax
import jax.numpy as jnp
from jax import lax
from jax.experimental import pallas as pl
from jax.experimental.pallas import tpu as pltpu
from jax.experimental.pallas import tpu_sc as plsc

_B = 64
_C = 384
_ROWS_PER_WORKER = 8
_NUM_ACTIVE = 16


def _sc_gather(y, gammas_table, betas_table):
    mesh = plsc.VectorSubcoreMesh(core_axis_name="c", subcore_axis_name="s")

    @functools.partial(
        pl.kernel,
        out_type=[
            jax.ShapeDtypeStruct((_B, _C), jnp.float32),
            jax.ShapeDtypeStruct((_B, _C), jnp.float32),
        ],
        mesh=mesh,
        scratch_types=[
            pltpu.VMEM((_ROWS_PER_WORKER,), jnp.int32),
            pltpu.VMEM((_ROWS_PER_WORKER, _C), jnp.float32),
            pltpu.SemaphoreType.DMA,
        ],
    )
    def gather_kernel(y_hbm, gt_hbm, bt_hbm, g_out, b_out, idx_v, rows_v, sem):
        wid = lax.axis_index("s") * 2 + lax.axis_index("c")
        base = (wid % 8) * _ROWS_PER_WORKER

        @pl.when(wid < 8)
        def _():
            pltpu.sync_copy(y_hbm.at[pl.ds(base, _ROWS_PER_WORKER)], idx_v)
            pltpu.async_copy(gt_hbm.at[idx_v], rows_v, sem).wait()
            pltpu.sync_copy(rows_v, g_out.at[pl.ds(base, _ROWS_PER_WORKER)])

        @pl.when((wid >= 8) & (wid < _NUM_ACTIVE))
        def _():
            pltpu.sync_copy(y_hbm.at[pl.ds(base, _ROWS_PER_WORKER)], idx_v)
            pltpu.async_copy(bt_hbm.at[idx_v], rows_v, sem).wait()
            pltpu.sync_copy(rows_v, b_out.at[pl.ds(base, _ROWS_PER_WORKER)])

    return gather_kernel(y, gammas_table, betas_table)


def _film_body(x_ref, g_ref, b_ref, o_ref):
    o_ref[...] = x_ref[...] * g_ref[...] + b_ref[...]


def _film_planes(xt, g, b, pb=98):
    p, bsz, c = xt.shape
    grid = (p // pb,)
    return pl.pallas_call(
        _film_body,
        grid=grid,
        in_specs=[
            pl.BlockSpec((pb, bsz, c), lambda i: (i, 0, 0)),
            pl.BlockSpec((bsz, c), lambda i: (0, 0)),
            pl.BlockSpec((bsz, c), lambda i: (0, 0)),
        ],
        out_specs=pl.BlockSpec((pb, bsz, c), lambda i: (i, 0, 0)),
        out_shape=jax.ShapeDtypeStruct((p, bsz, c), xt.dtype),
        compiler_params=pltpu.CompilerParams(
            dimension_semantics=("parallel",),
        ),
    )(xt, g, b)


def kernel(x, y, gammas_table, betas_table):
    g, b = _sc_gather(y.astype(jnp.int32), gammas_table, betas_table)
    bsz, c, h, w = x.shape
    xt = jnp.transpose(x, (2, 3, 0, 1)).reshape(h * w, bsz, c)
    ot = _film_planes(xt, g, b)
    out = jnp.transpose(ot.reshape(h, w, bsz, c), (2, 3, 0, 1))
    return (out, y)
